# Initial kernel scaffold; baseline (speedup 1.0000x reference)
import jax
import jax.numpy as jnp
from jax.experimental import pallas as pl


def kernel(x, edge_index, W1, b1, bn_gamma, bn_beta, bn_mean, bn_var, W2, b2):
    raise NotImplementedError("write your pallas kernel here")


# speedup vs baseline: 27.9714x; 27.9714x over previous
import functools

import jax
import jax.numpy as jnp
from jax import lax
from jax.experimental import pallas as pl
from jax.experimental.pallas import tpu as pltpu
from jax.experimental.pallas import tpu_sc as plsc

N = 10000
E = 320000
F = 128
C = 40
NC = 2
NS = 16
NW = NC * NS
EPT = E // NW
CH = 125
NCH = EPT // CH
NPT0 = 624
NPTL = N - (NS - 1) * NPT0
DEGW = 128


def _each_tile_rows(sid, make_copy):
    @pl.when(sid < NS - 1)
    def _():
        make_copy(sid * NPT0, NPT0)

    @pl.when(sid == NS - 1)
    def _():
        make_copy((NS - 1) * NPT0, NPTL)


def _sc_mesh():
    return plsc.VectorSubcoreMesh(
        core_axis_name="c", subcore_axis_name="s", num_cores=NC, num_subcores=NS
    )


def _deg_body(dstr, ones_h, zeros_h, out, idx_d, ones_v, acc):
    cid = lax.axis_index("c")
    sid = lax.axis_index("s")
    wid = cid * NS + sid
    pltpu.sync_copy(dstr.at[wid], idx_d)
    pltpu.sync_copy(ones_h, ones_v)
    _each_tile_rows(sid, lambda off, sz: pltpu.sync_copy(
        zeros_h.at[pl.ds(0, sz)], acc.at[pl.ds(off, sz)]))
    plsc.subcore_barrier()

    def body(c, carry):
        pltpu.sync_copy(ones_v, acc.at[idx_d.at[c]], add=True)
        return carry

    lax.fori_loop(0, NCH, body, 0)
    plsc.subcore_barrier()
    _each_tile_rows(sid, lambda off, sz: pltpu.sync_copy(
        acc.at[pl.ds(off, sz)], out.at[cid, pl.ds(off, sz)]))


def _make_deg():
    return pl.kernel(
        _deg_body,
        out_type=jax.ShapeDtypeStruct((NC, N, DEGW), jnp.float32),
        mesh=_sc_mesh(),
        scratch_types=[
            pltpu.VMEM((NCH, CH), jnp.int32),
            pltpu.VMEM((CH, DEGW), jnp.float32),
            pltpu.VMEM_SHARED((N, DEGW), jnp.float32),
        ],
    )


def _prop_body(y, ei, zeros_h, out, idxb, rows, acc,
               si0, si1, si2, si3, sr0, sr1):
    cid = lax.axis_index("c")
    sid = lax.axis_index("s")
    wid = cid * NS + sid
    sem_i = (si0, si1, si2, si3)
    sem_r = (sr0, sr1)

    def idx_start(c, slot):
        pltpu.async_copy(ei.at[wid, c], idxb.at[slot], sem_i[slot])

    def idx_wait(slot):
        pltpu.make_async_copy(ei.at[wid, 0], idxb.at[slot], sem_i[slot]).wait()

    def gather_start(slot, par):
        pltpu.async_copy(y.at[idxb.at[slot, 0]], rows.at[par], sem_r[par])

    def gather_wait(slot, par):
        pltpu.make_async_copy(y.at[idxb.at[slot, 0]], rows.at[par],
                              sem_r[par]).wait()

    def scatter(slot, par):
        pltpu.sync_copy(rows.at[par], acc.at[idxb.at[slot, 1]], add=True)

    _each_tile_rows(sid, lambda off, sz: pltpu.sync_copy(
        zeros_h.at[pl.ds(0, sz)], acc.at[pl.ds(off, sz)]))
    for s in range(4):
        idx_start(s, s)
    plsc.subcore_barrier()
    idx_wait(0)
    gather_start(0, 0)
    idx_wait(1)

    def body(i, carry):
        for b in range(4):
            c = i * 4 + b
            gather_start((b + 1) % 4, (b + 1) % 2)
            gather_wait(b, b % 2)
            scatter(b, b % 2)
            idx_start(c + 4, b)
            idx_wait((b + 2) % 4)
        return carry

    lax.fori_loop(0, (NCH - 4) // 4, body, 0)
    for c in range(NCH - 4, NCH):
        b = c % 4
        if c + 1 < NCH:
            gather_start((b + 1) % 4, (b + 1) % 2)
        gather_wait(b, b % 2)
        scatter(b, b % 2)
        if c + 2 < NCH:
            idx_wait((b + 2) % 4)

    plsc.subcore_barrier()
    _each_tile_rows(sid, lambda off, sz: pltpu.sync_copy(
        acc.at[pl.ds(off, sz)], out.at[cid, pl.ds(off, sz)]))


def _make_prop(d):
    return pl.kernel(
        _prop_body,
        out_type=jax.ShapeDtypeStruct((NC, N, d), jnp.float32),
        mesh=_sc_mesh(),
        scratch_types=[
            pltpu.VMEM((4, 2, CH), jnp.int32),
            pltpu.VMEM((2, CH, d), jnp.float32),
            pltpu.VMEM_SHARED((N, d), jnp.float32),
            pltpu.SemaphoreType.DMA,
            pltpu.SemaphoreType.DMA,
            pltpu.SemaphoreType.DMA,
            pltpu.SemaphoreType.DMA,
            pltpu.SemaphoreType.DMA,
            pltpu.SemaphoreType.DMA,
        ],
    )


_R = 1000


def _k2_body(x_ref, w1_ref, degp_ref, y_ref, dinv_ref):
    degp = degp_ref[...]
    deg = degp[0, :, 0:1] + degp[1, :, 0:1] + 1.0
    dinv = lax.rsqrt(deg)
    xw = jnp.dot(x_ref[...], w1_ref[...], preferred_element_type=jnp.float32)
    y_ref[...] = xw * dinv
    dinv_ref[...] = jnp.broadcast_to(dinv, dinv_ref.shape)


def _k4_body(p_ref, y1_ref, dinv_ref, g_ref, bta_ref, mu_ref, var_ref, b1_ref,
             w2_ref, y2_ref):
    s = dinv_ref[...][:, 0:1]
    p = p_ref[...]
    h = (p[0] + p[1] + y1_ref[...]) * s + b1_ref[...]
    h = g_ref[...] * (h - mu_ref[...]) * lax.rsqrt(var_ref[...] + 1e-5) + bta_ref[...]
    h = jnp.maximum(h, 0.0)
    m2 = jnp.dot(h, w2_ref[...], preferred_element_type=jnp.float32) * s
    y2_ref[...] = jnp.concatenate(
        [m2, jnp.zeros((m2.shape[0], F - C), jnp.float32)], axis=1)


def _k6_body(q_ref, y2_ref, dinv_ref, b2_ref, out_ref):
    s = dinv_ref[...][:, 0:1]
    q = q_ref[...]
    t = (q[0] + q[1] + y2_ref[...])[:, :C]
    out_ref[...] = t * s + b2_ref[...]


def _make_k2():
    return pl.pallas_call(
        _k2_body,
        grid=(N // _R,),
        in_specs=[
            pl.BlockSpec((_R, F), lambda i: (i, 0)),
            pl.BlockSpec((F, F), lambda i: (0, 0)),
            pl.BlockSpec((NC, _R, DEGW), lambda i: (0, i, 0)),
        ],
        out_specs=[
            pl.BlockSpec((_R, F), lambda i: (i, 0)),
            pl.BlockSpec((_R, DEGW), lambda i: (i, 0)),
        ],
        out_shape=[
            jax.ShapeDtypeStruct((N, F), jnp.float32),
            jax.ShapeDtypeStruct((N, DEGW), jnp.float32),
        ],
    )


def _make_k4():
    return pl.pallas_call(
        _k4_body,
        grid=(N // _R,),
        in_specs=[
            pl.BlockSpec((NC, _R, F), lambda i: (0, i, 0)),
            pl.BlockSpec((_R, F), lambda i: (i, 0)),
            pl.BlockSpec((_R, DEGW), lambda i: (i, 0)),
            pl.BlockSpec((1, F), lambda i: (0, 0)),
            pl.BlockSpec((1, F), lambda i: (0, 0)),
            pl.BlockSpec((1, F), lambda i: (0, 0)),
            pl.BlockSpec((1, F), lambda i: (0, 0)),
            pl.BlockSpec((1, F), lambda i: (0, 0)),
            pl.BlockSpec((F, C), lambda i: (0, 0)),
        ],
        out_specs=pl.BlockSpec((_R, F), lambda i: (i, 0)),
        out_shape=jax.ShapeDtypeStruct((N, F), jnp.float32),
    )


def _make_k6():
    return pl.pallas_call(
        _k6_body,
        grid=(N // _R,),
        in_specs=[
            pl.BlockSpec((NC, _R, F), lambda i: (0, i, 0)),
            pl.BlockSpec((_R, F), lambda i: (i, 0)),
            pl.BlockSpec((_R, DEGW), lambda i: (i, 0)),
            pl.BlockSpec((1, C), lambda i: (0, 0)),
        ],
        out_specs=pl.BlockSpec((_R, C), lambda i: (i, 0)),
        out_shape=jax.ShapeDtypeStruct((N, C), jnp.float32),
    )


def kernel(x, edge_index, W1, b1, bn_gamma, bn_beta, bn_mean, bn_var, W2, b2):
    src = edge_index[0].reshape(NW, NCH, CH)
    dst = edge_index[1].reshape(NW, NCH, CH)
    ei = jnp.stack([src, dst], axis=2)
    ones8 = jnp.ones((CH, DEGW), jnp.float32)
    zeros8 = jnp.zeros((NPTL, DEGW), jnp.float32)
    zerosf = jnp.zeros((NPTL, F), jnp.float32)

    degp = _make_deg()(dst, ones8, zeros8)
    y1, dinv = _make_k2()(x, W1, degp)
    p = _make_prop(F)(y1, ei, zerosf)
    y2 = _make_k4()(p, y1, dinv, bn_gamma.reshape(1, F), bn_beta.reshape(1, F),
                    bn_mean.reshape(1, F), bn_var.reshape(1, F),
                    b1.reshape(1, F), W2)
    q = _make_prop(F)(y2, ei, zerosf)
    return _make_k6()(q, y2, dinv, b2.reshape(1, C))
